# Initial kernel scaffold; baseline (speedup 1.0000x reference)
import jax
import jax.numpy as jnp
from jax.experimental import pallas as pl


def kernel(indices, codebook):
    raise NotImplementedError("write your pallas kernel here")


# speedup vs baseline: 2.0430x; 2.0430x over previous
import jax
import jax.numpy as jnp
from jax import lax
from jax.experimental import pallas as pl
from jax.experimental.pallas import tpu as pltpu

_TT = 256


def _body(idx_ref, cb_ref, out_ref, scratch_ref):
    def tok(i, carry):
        row = idx_ref[0, 0, i]
        scratch_ref[pl.ds(i, 1), :] = cb_ref[pl.ds(row, 1), :]
        return carry

    lax.fori_loop(0, _TT, tok, 0, unroll=8)
    out_ref[0] = scratch_ref[...].T


def kernel(indices, codebook):
    B, T = indices.shape
    V, D = codebook.shape
    NT = T // _TT
    idx = jnp.clip(indices.astype(jnp.int32), 0, V - 1)
    idx = idx.reshape(B * NT, 1, _TT)

    out = pl.pallas_call(
        _body,
        grid=(B, NT),
        in_specs=[
            pl.BlockSpec((1, 1, _TT), lambda b, t: (b * NT + t, 0, 0),
                         memory_space=pltpu.SMEM),
            pl.BlockSpec((V, D), lambda b, t: (0, 0)),
        ],
        out_specs=pl.BlockSpec((1, D, _TT), lambda b, t: (b, 0, t)),
        out_shape=jax.ShapeDtypeStruct((B, D, T), jnp.float32),
        scratch_shapes=[pltpu.VMEM((_TT, D), jnp.float32)],
    )(idx, codebook)
    return out


# speedup vs baseline: 2.1477x; 1.0513x over previous
import jax
import jax.numpy as jnp
from jax import lax
from jax.experimental import pallas as pl
from jax.experimental.pallas import tpu as pltpu

_TT = 256


def _body(idx_ref, cb_ref, out_ref, scratch_ref):
    def tok(i, carry):
        row = idx_ref[0, 0, i]
        scratch_ref[pl.ds(i, 1), :, :] = cb_ref[pl.ds(row, 1), :, :]
        return carry

    lax.fori_loop(0, _TT, tok, 0, unroll=8)
    for s in range(8):
        out_ref[0, pl.ds(128 * s, 128), :] = scratch_ref[:, s, :].T


def kernel(indices, codebook):
    B, T = indices.shape
    V, D = codebook.shape
    NT = T // _TT
    idx = jnp.clip(indices.astype(jnp.int32), 0, V - 1)
    idx = idx.reshape(B * NT, 1, _TT)
    cb3 = codebook.reshape(V, 8, D // 8)

    out = pl.pallas_call(
        _body,
        grid=(B, NT),
        in_specs=[
            pl.BlockSpec((1, 1, _TT), lambda b, t: (b * NT + t, 0, 0),
                         memory_space=pltpu.SMEM),
            pl.BlockSpec((V, 8, D // 8), lambda b, t: (0, 0, 0)),
        ],
        out_specs=pl.BlockSpec((1, D, _TT), lambda b, t: (b, 0, t)),
        out_shape=jax.ShapeDtypeStruct((B, D, T), jnp.float32),
        scratch_shapes=[pltpu.VMEM((_TT, 8, D // 8), jnp.float32)],
    )(idx, cb3)
    return out
